# baseline (speedup 1.0000x reference)
import functools

import jax
import jax.numpy as jnp
from jax import lax
from jax.experimental import pallas as pl
from jax.experimental.pallas import tpu as pltpu
from jax.experimental.pallas import tpu_sc as plsc

N_NODES = 10000
N_PAD = 10240
N_CORES = 2
N_SUBCORES = 16
N_WORKERS = N_CORES * N_SUBCORES
LANES = 16
CH = 128

_GDN = lax.GatherDimensionNumbers(
    offset_dims=(), collapsed_slice_dims=(0,), start_index_map=(0,))


def _lane_perm(x, idx):
    return lax.gather(x, idx[:, None], _GDN, (1,),
                      mode=lax.GatherScatterMode.PROMISE_IN_BOUNDS)


def _lane_allsum(x, lane):
    for k in (8, 4, 2, 1):
        x = x + _lane_perm(x, lane ^ k)
    return x


def _edge_messages(bt, ef_t, W_time, b_time, W_ef, W_tf, b_edge, tile_e, e_pad):
    E = bt.shape[0]
    F = ef_t.shape[0]
    H = W_tf.shape[1]
    last = (E - 1) // tile_e

    def body(bt_ref, ef_ref, wt_ref, btm_ref, wef_ref, wtf_ref, be_ref, out_ref):
        tf = jnp.maximum(bt_ref[...][:, None] * wt_ref[...] + btm_ref[...], 0.0)
        acc = lax.dot_general(ef_ref[...], wef_ref[...],
                              (((0,), (0,)), ((), ())),
                              preferred_element_type=jnp.float32)
        acc = acc + jnp.dot(tf, wtf_ref[...], preferred_element_type=jnp.float32)
        out_ref[...] = jnp.maximum(acc + be_ref[...], 0.0)

    clamp = lambda i: jnp.minimum(i, last)
    return pl.pallas_call(
        body,
        grid=(e_pad // tile_e,),
        in_specs=[
            pl.BlockSpec((tile_e,), lambda i: (clamp(i),)),
            pl.BlockSpec((F, tile_e), lambda i: (0, clamp(i))),
            pl.BlockSpec((1, H), lambda i: (0, 0)),
            pl.BlockSpec((1, H), lambda i: (0, 0)),
            pl.BlockSpec((F, H), lambda i: (0, 0)),
            pl.BlockSpec((H, H), lambda i: (0, 0)),
            pl.BlockSpec((1, H), lambda i: (0, 0)),
        ],
        out_specs=pl.BlockSpec((tile_e, H), lambda i: (clamp(i), 0)),
        out_shape=jax.ShapeDtypeStruct((e_pad, H), jnp.float32),
    )(bt, ef_t, W_time, b_time, W_ef, W_tf, b_edge)


def _segment_sum_sc(cf, dst, zeros):
    E, H = cf.shape
    N = zeros.shape[0]
    n_chunks = E // CH
    chunks_per_w = n_chunks // N_WORKERS
    assert chunks_per_w * N_WORKERS == n_chunks and chunks_per_w % 2 == 0
    n_pairs = chunks_per_w // 2
    rows_per_sub = N // N_SUBCORES

    mesh = plsc.VectorSubcoreMesh(core_axis_name="c", subcore_axis_name="s")

    @functools.partial(
        pl.kernel,
        out_type=jax.ShapeDtypeStruct((N_CORES * N, H), jnp.float32),
        mesh=mesh,
        scratch_types=[
            pltpu.VMEM((2, CH), jnp.int32),
            pltpu.VMEM((2, CH, H), jnp.float32),
            pltpu.VMEM_SHARED((N, H), jnp.float32),
            pltpu.SemaphoreType.DMA,
            pltpu.SemaphoreType.DMA,
        ],
    )
    def k(cf_hbm, dst_hbm, z_hbm, out_hbm, idx_v, rows_v, h_sh, sem0, sem1):
        c = lax.axis_index("c")
        s = lax.axis_index("s")
        w = s * N_CORES + c
        sems = (sem0, sem1)

        sub_lo = s * rows_per_sub
        pltpu.sync_copy(z_hbm.at[pl.ds(sub_lo, rows_per_sub)],
                        h_sh.at[pl.ds(sub_lo, rows_per_sub)])
        plsc.subcore_barrier()

        def load_issue(ci, b):
            base = (w + ci * N_WORKERS) * CH
            pltpu.async_copy(dst_hbm.at[pl.ds(base, CH)], idx_v.at[b], sems[b])
            pltpu.async_copy(cf_hbm.at[pl.ds(base, CH)], rows_v.at[b], sems[b])

        def load_wait(b):
            pltpu.make_async_copy(dst_hbm.at[pl.ds(0, CH)], idx_v.at[b], sems[b]).wait()
            pltpu.make_async_copy(cf_hbm.at[pl.ds(0, CH)], rows_v.at[b], sems[b]).wait()

        def scatter(b):
            pltpu.sync_copy(rows_v.at[b], h_sh.at[idx_v.at[b]], add=True)

        load_issue(0, 0)

        def body(j, carry):
            k0 = 2 * j
            load_wait(0)
            load_issue(k0 + 1, 1)
            scatter(0)
            load_wait(1)

            @pl.when(k0 + 2 < chunks_per_w)
            def _():
                load_issue(k0 + 2, 0)

            scatter(1)
            return carry

        lax.fori_loop(0, n_pairs, body, 0)
        plsc.subcore_barrier()
        pltpu.sync_copy(h_sh.at[pl.ds(sub_lo, rows_per_sub)],
                        out_hbm.at[pl.ds(c * N + sub_lo, rows_per_sub)])

    return k(cf, dst, zeros)


def _node_embeddings(hp, hq, W_src, b_src, W_dst, b_dst, tile_n):
    _, N, H = hp.shape

    def body(hp_ref, hq_ref, ws_ref, bs_ref, wd_ref, bd_ref, a_ref, b_ref):
        h = hp_ref[0] + hp_ref[1] + hq_ref[0] + hq_ref[1]
        a = jnp.dot(h, ws_ref[...], preferred_element_type=jnp.float32) + bs_ref[...]
        b = jnp.dot(h, wd_ref[...], preferred_element_type=jnp.float32) + bd_ref[...]
        a_ref[...] = a
        b_ref[...] = b

    return pl.pallas_call(
        body,
        grid=(N // tile_n,),
        in_specs=[
            pl.BlockSpec((2, tile_n, H), lambda i: (0, i, 0)),
            pl.BlockSpec((2, tile_n, H), lambda i: (0, i, 0)),
            pl.BlockSpec((H, H), lambda i: (0, 0)),
            pl.BlockSpec((1, H), lambda i: (0, 0)),
            pl.BlockSpec((H, H), lambda i: (0, 0)),
            pl.BlockSpec((1, H), lambda i: (0, 0)),
        ],
        out_specs=[
            pl.BlockSpec((tile_n, H), lambda i: (i, 0)),
            pl.BlockSpec((tile_n, H), lambda i: (i, 0)),
        ],
        out_shape=[
            jax.ShapeDtypeStruct((N, H), jnp.float32),
            jax.ShapeDtypeStruct((N, H), jnp.float32),
        ],
    )(hp, hq, W_src, b_src, W_dst, b_dst)


def _predict_sc(A, Bm, s_idx, p_idx, n_idx, w_vec, bout):
    N, H = A.shape
    Bpad = s_idx.shape[0]
    n_chunks = Bpad // CH
    NG = H // LANES
    per_pair = n_chunks // N_SUBCORES
    nc0 = (per_pair * 42 // 50) // 2 * 2
    nc1 = per_pair - nc0
    assert nc0 >= 2 and nc1 >= 2 and nc1 % 2 == 0
    assert (nc0 + nc1) * N_SUBCORES == n_chunks

    mesh = plsc.VectorSubcoreMesh(core_axis_name="c", subcore_axis_name="s")

    @functools.partial(
        pl.kernel,
        out_type=(
            jax.ShapeDtypeStruct((Bpad,), jnp.float32),
            jax.ShapeDtypeStruct((Bpad,), jnp.float32),
        ),
        mesh=mesh,
        scratch_types=[
            pltpu.VMEM((2, CH), jnp.int32),
            pltpu.VMEM((2, CH), jnp.int32),
            pltpu.VMEM((2, CH), jnp.int32),
            pltpu.VMEM((2, CH, H), jnp.float32),
            pltpu.VMEM((2, CH, H), jnp.float32),
            pltpu.VMEM((2, CH, H), jnp.float32),
            pltpu.VMEM((2, CH), jnp.float32),
            pltpu.VMEM((2, CH), jnp.float32),
            pltpu.VMEM((H,), jnp.float32),
            pltpu.VMEM((LANES,), jnp.float32),
            pltpu.SemaphoreType.DMA,
            pltpu.SemaphoreType.DMA,
            pltpu.SemaphoreType.DMA,
            pltpu.SemaphoreType.DMA,
            pltpu.SemaphoreType.DMA,
            pltpu.SemaphoreType.DMA,
        ],
    )
    def k(a_hbm, b_hbm, s_hbm, p_hbm, n_hbm, w_hbm, bo_hbm,
          pos_hbm, neg_hbm,
          si_v, pi_v, ni_v, as_v, bp_v, bn_v, po_v, ne_v, w_v, bo_v,
          smi0, smi1, smg0, smg1, sms0, sms1):
        c = lax.axis_index("c")
        sc = lax.axis_index("s")
        sem_i = (smi0, smi1)
        sem_g = (smg0, smg1)
        sem_s = (sms0, sms1)
        pltpu.sync_copy(w_hbm, w_v)
        pltpu.sync_copy(bo_hbm, bo_v)
        wregs = [w_v[pl.ds(g * LANES, LANES)] for g in range(NG)]
        bo = bo_v[...]
        lane = lax.iota(jnp.int32, LANES)

        n_my = jnp.where(c == 0, nc0, nc1)
        first_chunk = jnp.where(c == 0, sc * nc0, N_SUBCORES * nc0 + sc * nc1)

        def chunk_base(ci):
            return (first_chunk + ci) * CH

        def idx_issue(ci, b):
            base = chunk_base(ci)
            pltpu.async_copy(s_hbm.at[pl.ds(base, CH)], si_v.at[b], sem_i[b])
            pltpu.async_copy(p_hbm.at[pl.ds(base, CH)], pi_v.at[b], sem_i[b])
            pltpu.async_copy(n_hbm.at[pl.ds(base, CH)], ni_v.at[b], sem_i[b])

        def idx_wait(b):
            pltpu.make_async_copy(s_hbm.at[pl.ds(0, CH)], si_v.at[b], sem_i[b]).wait()
            pltpu.make_async_copy(p_hbm.at[pl.ds(0, CH)], pi_v.at[b], sem_i[b]).wait()
            pltpu.make_async_copy(n_hbm.at[pl.ds(0, CH)], ni_v.at[b], sem_i[b]).wait()

        def gat_issue(b):
            pltpu.async_copy(a_hbm.at[si_v.at[b]], as_v.at[b], sem_g[b])
            pltpu.async_copy(b_hbm.at[pi_v.at[b]], bp_v.at[b], sem_g[b])
            pltpu.async_copy(b_hbm.at[ni_v.at[b]], bn_v.at[b], sem_g[b])

        def gat_wait(b):
            pltpu.make_async_copy(a_hbm.at[si_v.at[b]], as_v.at[b], sem_g[b]).wait()
            pltpu.make_async_copy(b_hbm.at[pi_v.at[b]], bp_v.at[b], sem_g[b]).wait()
            pltpu.make_async_copy(b_hbm.at[ni_v.at[b]], bn_v.at[b], sem_g[b]).wait()

        def out_wait(b):
            pltpu.make_async_copy(po_v.at[b], pos_hbm.at[pl.ds(0, CH)], sem_s[b]).wait()
            pltpu.make_async_copy(ne_v.at[b], neg_hbm.at[pl.ds(0, CH)], sem_s[b]).wait()

        def compute(ci, b):
            @pl.when(ci >= 2)
            def _():
                out_wait(b)

            def grp_body(g, carry):
                def row_body(r, acc):
                    pacc, nacc = acc
                    row = g * LANES + r
                    pa = jnp.zeros((LANES,), jnp.float32)
                    na = jnp.zeros((LANES,), jnp.float32)
                    for fg in range(NG):
                        sl = pl.ds(fg * LANES, LANES)
                        av = as_v[b, row, sl]
                        pa = pa + jnp.maximum(av + bp_v[b, row, sl], 0.0) * wregs[fg]
                        na = na + jnp.maximum(av + bn_v[b, row, sl], 0.0) * wregs[fg]
                    pacc = jnp.where(lane == r, _lane_allsum(pa, lane), pacc)
                    nacc = jnp.where(lane == r, _lane_allsum(na, lane), nacc)
                    return pacc, nacc

                z = jnp.zeros((LANES,), jnp.float32)
                pacc, nacc = lax.fori_loop(0, LANES, row_body, (z, z))
                po_v[b, pl.ds(g * LANES, LANES)] = pacc + bo
                ne_v[b, pl.ds(g * LANES, LANES)] = nacc + bo
                return carry

            lax.fori_loop(0, CH // LANES, grp_body, 0)
            base = chunk_base(ci)
            pltpu.async_copy(po_v.at[b], pos_hbm.at[pl.ds(base, CH)], sem_s[b])
            pltpu.async_copy(ne_v.at[b], neg_hbm.at[pl.ds(base, CH)], sem_s[b])

        def step(ci, b, nb):
            gat_wait(b)

            @pl.when(ci + 1 < n_my)
            def _():
                idx_wait(nb)
                gat_issue(nb)

            @pl.when(ci + 2 < n_my)
            def _():
                idx_issue(ci + 2, b)

            compute(ci, b)

        base0 = chunk_base(0)
        pltpu.sync_copy(s_hbm.at[pl.ds(base0, CH)], si_v.at[0])
        pltpu.sync_copy(p_hbm.at[pl.ds(base0, CH)], pi_v.at[0])
        pltpu.sync_copy(n_hbm.at[pl.ds(base0, CH)], ni_v.at[0])
        gat_issue(0)
        idx_issue(1, 1)

        def body(j, carry):
            step(2 * j, 0, 1)
            step(2 * j + 1, 1, 0)
            return carry

        lax.fori_loop(0, n_my // 2, body, 0)
        out_wait(0)
        out_wait(1)

    return k(A, Bm, s_idx, p_idx, n_idx, w_vec, bout)


def kernel(edge_index, ef, bt, s, p, n,
           W_time, b_time, W_edge, b_edge,
           W_src, b_src, W_dst, b_dst, W_out, b_out,
           neg_samples=1):
    H = W_src.shape[0]
    F = ef.shape[1]
    B = s.shape[0]
    E = bt.shape[0]

    dst = edge_index[1]
    W_ef = W_edge[:F]
    W_tf = W_edge[F:]

    EH = E // 2
    e_quantum = N_WORKERS * 2 * CH
    E_pad = ((EH + e_quantum - 1) // e_quantum) * e_quantum
    ef_t = ef.T
    zeros = jnp.zeros((N_PAD, H), jnp.float32)
    hps = []
    for lo in (0, EH):
        dst_h = jnp.pad(lax.dynamic_slice_in_dim(dst, lo, EH),
                        (0, E_pad - EH), constant_values=N_NODES)
        cf_h = _edge_messages(lax.dynamic_slice_in_dim(bt, lo, EH),
                              lax.dynamic_slice_in_dim(ef_t, lo, EH, axis=1),
                              W_time, b_time.reshape(1, H),
                              W_ef, W_tf, b_edge.reshape(1, H),
                              tile_e=2048, e_pad=E_pad)
        hps.append(_segment_sum_sc(cf_h, dst_h, zeros))

    A, Bm = _node_embeddings(hps[0].reshape(2, N_PAD, H),
                             hps[1].reshape(2, N_PAD, H),
                             W_src, b_src.reshape(1, H),
                             W_dst, b_dst.reshape(1, H), tile_n=2048)

    quantum = N_WORKERS * CH
    Bpad = ((B + quantum - 1) // quantum) * quantum
    pad = Bpad - B
    sp = jnp.pad(s, (0, pad))
    pp = jnp.pad(p, (0, pad))
    np_ = jnp.pad(n, (0, pad))

    posf, negf = _predict_sc(A, Bm, sp, pp, np_,
                             W_out.reshape(H), jnp.broadcast_to(b_out, (LANES,)))
    return posf[:B, None], negf[:B, None]


# speedup vs baseline: 1.0100x; 1.0100x over previous
import functools

import jax
import jax.numpy as jnp
from jax import lax
from jax.experimental import pallas as pl
from jax.experimental.pallas import tpu as pltpu
from jax.experimental.pallas import tpu_sc as plsc

N_NODES = 10000
N_PAD = 10240
N_CORES = 2
N_SUBCORES = 16
N_WORKERS = N_CORES * N_SUBCORES
LANES = 16
CH = 128

_GDN = lax.GatherDimensionNumbers(
    offset_dims=(), collapsed_slice_dims=(0,), start_index_map=(0,))


def _lane_perm(x, idx):
    return lax.gather(x, idx[:, None], _GDN, (1,),
                      mode=lax.GatherScatterMode.PROMISE_IN_BOUNDS)


def _lane_allsum(x, lane):
    for k in (8, 4, 2, 1):
        x = x + _lane_perm(x, lane ^ k)
    return x


def _edge_messages(bt, ef_t, W_time, b_time, W_ef, W_tf, b_edge, tile_e, e_pad):
    E = bt.shape[0]
    F = ef_t.shape[0]
    H = W_tf.shape[1]
    last = (E - 1) // tile_e

    def body(bt_ref, ef_ref, wt_ref, btm_ref, wef_ref, wtf_ref, be_ref, out_ref):
        tf = jnp.maximum(bt_ref[...][:, None] * wt_ref[...] + btm_ref[...], 0.0)
        acc = lax.dot_general(ef_ref[...], wef_ref[...],
                              (((0,), (0,)), ((), ())),
                              preferred_element_type=jnp.float32)
        acc = acc + jnp.dot(tf, wtf_ref[...], preferred_element_type=jnp.float32)
        out_ref[...] = jnp.maximum(acc + be_ref[...], 0.0)

    clamp = lambda i: jnp.minimum(i, last)
    return pl.pallas_call(
        body,
        grid=(e_pad // tile_e,),
        in_specs=[
            pl.BlockSpec((tile_e,), lambda i: (clamp(i),)),
            pl.BlockSpec((F, tile_e), lambda i: (0, clamp(i))),
            pl.BlockSpec((1, H), lambda i: (0, 0)),
            pl.BlockSpec((1, H), lambda i: (0, 0)),
            pl.BlockSpec((F, H), lambda i: (0, 0)),
            pl.BlockSpec((H, H), lambda i: (0, 0)),
            pl.BlockSpec((1, H), lambda i: (0, 0)),
        ],
        out_specs=pl.BlockSpec((tile_e, H), lambda i: (clamp(i), 0)),
        out_shape=jax.ShapeDtypeStruct((e_pad, H), jnp.float32),
    )(bt, ef_t, W_time, b_time, W_ef, W_tf, b_edge)


def _segment_sum_sc(cf, dst, zeros):
    E, H = cf.shape
    N = zeros.shape[0]
    n_chunks = E // CH
    chunks_per_w = n_chunks // N_WORKERS
    assert chunks_per_w * N_WORKERS == n_chunks and chunks_per_w % 2 == 0
    n_pairs = chunks_per_w // 2
    rows_per_sub = N // N_SUBCORES

    mesh = plsc.VectorSubcoreMesh(core_axis_name="c", subcore_axis_name="s")

    @functools.partial(
        pl.kernel,
        out_type=jax.ShapeDtypeStruct((N_CORES * N, H), jnp.float32),
        mesh=mesh,
        scratch_types=[
            pltpu.VMEM((2, CH), jnp.int32),
            pltpu.VMEM((2, CH, H), jnp.float32),
            pltpu.VMEM_SHARED((N, H), jnp.float32),
            pltpu.SemaphoreType.DMA,
            pltpu.SemaphoreType.DMA,
        ],
    )
    def k(cf_hbm, dst_hbm, z_hbm, out_hbm, idx_v, rows_v, h_sh, sem0, sem1):
        c = lax.axis_index("c")
        s = lax.axis_index("s")
        w = s * N_CORES + c
        sems = (sem0, sem1)

        sub_lo = s * rows_per_sub
        pltpu.sync_copy(z_hbm.at[pl.ds(sub_lo, rows_per_sub)],
                        h_sh.at[pl.ds(sub_lo, rows_per_sub)])
        plsc.subcore_barrier()

        def load_issue(ci, b):
            base = (w + ci * N_WORKERS) * CH
            pltpu.async_copy(dst_hbm.at[pl.ds(base, CH)], idx_v.at[b], sems[b])
            pltpu.async_copy(cf_hbm.at[pl.ds(base, CH)], rows_v.at[b], sems[b])

        def load_wait(b):
            pltpu.make_async_copy(dst_hbm.at[pl.ds(0, CH)], idx_v.at[b], sems[b]).wait()
            pltpu.make_async_copy(cf_hbm.at[pl.ds(0, CH)], rows_v.at[b], sems[b]).wait()

        def scatter(b):
            pltpu.sync_copy(rows_v.at[b], h_sh.at[idx_v.at[b]], add=True)

        load_issue(0, 0)

        def body(j, carry):
            k0 = 2 * j
            load_wait(0)
            load_issue(k0 + 1, 1)
            scatter(0)
            load_wait(1)

            @pl.when(k0 + 2 < chunks_per_w)
            def _():
                load_issue(k0 + 2, 0)

            scatter(1)
            return carry

        lax.fori_loop(0, n_pairs, body, 0)
        plsc.subcore_barrier()
        pltpu.sync_copy(h_sh.at[pl.ds(sub_lo, rows_per_sub)],
                        out_hbm.at[pl.ds(c * N + sub_lo, rows_per_sub)])

    return k(cf, dst, zeros)


def _node_embeddings(hp, W_src, b_src, W_dst, b_dst, tile_n):
    _, N, H = hp.shape

    def body(hp_ref, ws_ref, bs_ref, wd_ref, bd_ref, a_ref, b_ref):
        h = hp_ref[0] + hp_ref[1]
        a = jnp.dot(h, ws_ref[...], preferred_element_type=jnp.float32) + bs_ref[...]
        b = jnp.dot(h, wd_ref[...], preferred_element_type=jnp.float32) + bd_ref[...]
        a_ref[...] = a
        b_ref[...] = b

    return pl.pallas_call(
        body,
        grid=(N // tile_n,),
        in_specs=[
            pl.BlockSpec((2, tile_n, H), lambda i: (0, i, 0)),
            pl.BlockSpec((H, H), lambda i: (0, 0)),
            pl.BlockSpec((1, H), lambda i: (0, 0)),
            pl.BlockSpec((H, H), lambda i: (0, 0)),
            pl.BlockSpec((1, H), lambda i: (0, 0)),
        ],
        out_specs=[
            pl.BlockSpec((tile_n, H), lambda i: (i, 0)),
            pl.BlockSpec((tile_n, H), lambda i: (i, 0)),
        ],
        out_shape=[
            jax.ShapeDtypeStruct((N, H), jnp.float32),
            jax.ShapeDtypeStruct((N, H), jnp.float32),
        ],
    )(hp, W_src, b_src, W_dst, b_dst)


def _predict_sc(A, Bm, s_idx, p_idx, n_idx, w_vec, bout):
    N, H = A.shape
    Bpad = s_idx.shape[0]
    n_chunks = Bpad // CH
    NG = H // LANES
    per_pair = n_chunks // N_SUBCORES
    nc0 = (per_pair * 42 // 50) // 2 * 2
    nc1 = per_pair - nc0
    assert nc0 >= 2 and nc1 >= 2 and nc1 % 2 == 0
    assert (nc0 + nc1) * N_SUBCORES == n_chunks

    mesh = plsc.VectorSubcoreMesh(core_axis_name="c", subcore_axis_name="s")

    @functools.partial(
        pl.kernel,
        out_type=(
            jax.ShapeDtypeStruct((Bpad,), jnp.float32),
            jax.ShapeDtypeStruct((Bpad,), jnp.float32),
        ),
        mesh=mesh,
        scratch_types=[
            pltpu.VMEM((2, CH), jnp.int32),
            pltpu.VMEM((2, CH), jnp.int32),
            pltpu.VMEM((2, CH), jnp.int32),
            pltpu.VMEM((2, CH, H), jnp.float32),
            pltpu.VMEM((2, CH, H), jnp.float32),
            pltpu.VMEM((2, CH, H), jnp.float32),
            pltpu.VMEM((2, CH), jnp.float32),
            pltpu.VMEM((2, CH), jnp.float32),
            pltpu.VMEM((H,), jnp.float32),
            pltpu.VMEM((LANES,), jnp.float32),
            pltpu.SemaphoreType.DMA,
            pltpu.SemaphoreType.DMA,
            pltpu.SemaphoreType.DMA,
            pltpu.SemaphoreType.DMA,
            pltpu.SemaphoreType.DMA,
            pltpu.SemaphoreType.DMA,
        ],
    )
    def k(a_hbm, b_hbm, s_hbm, p_hbm, n_hbm, w_hbm, bo_hbm,
          pos_hbm, neg_hbm,
          si_v, pi_v, ni_v, as_v, bp_v, bn_v, po_v, ne_v, w_v, bo_v,
          smi0, smi1, smg0, smg1, sms0, sms1):
        c = lax.axis_index("c")
        sc = lax.axis_index("s")
        sem_i = (smi0, smi1)
        sem_g = (smg0, smg1)
        sem_s = (sms0, sms1)
        pltpu.sync_copy(w_hbm, w_v)
        pltpu.sync_copy(bo_hbm, bo_v)
        wregs = [w_v[pl.ds(g * LANES, LANES)] for g in range(NG)]
        bo = bo_v[...]
        lane = lax.iota(jnp.int32, LANES)

        n_my = jnp.where(c == 0, nc0, nc1)
        first_chunk = jnp.where(c == 0, sc * nc0, N_SUBCORES * nc0 + sc * nc1)

        def chunk_base(ci):
            return (first_chunk + ci) * CH

        def idx_issue(ci, b):
            base = chunk_base(ci)
            pltpu.async_copy(s_hbm.at[pl.ds(base, CH)], si_v.at[b], sem_i[b])
            pltpu.async_copy(p_hbm.at[pl.ds(base, CH)], pi_v.at[b], sem_i[b])
            pltpu.async_copy(n_hbm.at[pl.ds(base, CH)], ni_v.at[b], sem_i[b])

        def idx_wait(b):
            pltpu.make_async_copy(s_hbm.at[pl.ds(0, CH)], si_v.at[b], sem_i[b]).wait()
            pltpu.make_async_copy(p_hbm.at[pl.ds(0, CH)], pi_v.at[b], sem_i[b]).wait()
            pltpu.make_async_copy(n_hbm.at[pl.ds(0, CH)], ni_v.at[b], sem_i[b]).wait()

        def gat_issue(b):
            pltpu.async_copy(a_hbm.at[si_v.at[b]], as_v.at[b], sem_g[b])
            pltpu.async_copy(b_hbm.at[pi_v.at[b]], bp_v.at[b], sem_g[b])
            pltpu.async_copy(b_hbm.at[ni_v.at[b]], bn_v.at[b], sem_g[b])

        def gat_wait(b):
            pltpu.make_async_copy(a_hbm.at[si_v.at[b]], as_v.at[b], sem_g[b]).wait()
            pltpu.make_async_copy(b_hbm.at[pi_v.at[b]], bp_v.at[b], sem_g[b]).wait()
            pltpu.make_async_copy(b_hbm.at[ni_v.at[b]], bn_v.at[b], sem_g[b]).wait()

        def out_wait(b):
            pltpu.make_async_copy(po_v.at[b], pos_hbm.at[pl.ds(0, CH)], sem_s[b]).wait()
            pltpu.make_async_copy(ne_v.at[b], neg_hbm.at[pl.ds(0, CH)], sem_s[b]).wait()

        def compute(ci, b):
            @pl.when(ci >= 2)
            def _():
                out_wait(b)

            def grp_body(g, carry):
                def row_body(r, acc):
                    pacc, nacc = acc
                    row = g * LANES + r
                    pa = jnp.zeros((LANES,), jnp.float32)
                    na = jnp.zeros((LANES,), jnp.float32)
                    for fg in range(NG):
                        sl = pl.ds(fg * LANES, LANES)
                        av = as_v[b, row, sl]
                        pa = pa + jnp.maximum(av + bp_v[b, row, sl], 0.0) * wregs[fg]
                        na = na + jnp.maximum(av + bn_v[b, row, sl], 0.0) * wregs[fg]
                    pacc = jnp.where(lane == r, _lane_allsum(pa, lane), pacc)
                    nacc = jnp.where(lane == r, _lane_allsum(na, lane), nacc)
                    return pacc, nacc

                z = jnp.zeros((LANES,), jnp.float32)
                pacc, nacc = lax.fori_loop(0, LANES, row_body, (z, z))
                po_v[b, pl.ds(g * LANES, LANES)] = pacc + bo
                ne_v[b, pl.ds(g * LANES, LANES)] = nacc + bo
                return carry

            lax.fori_loop(0, CH // LANES, grp_body, 0)
            base = chunk_base(ci)
            pltpu.async_copy(po_v.at[b], pos_hbm.at[pl.ds(base, CH)], sem_s[b])
            pltpu.async_copy(ne_v.at[b], neg_hbm.at[pl.ds(base, CH)], sem_s[b])

        def step(ci, b, nb):
            gat_wait(b)

            @pl.when(ci + 1 < n_my)
            def _():
                idx_wait(nb)
                gat_issue(nb)

            @pl.when(ci + 2 < n_my)
            def _():
                idx_issue(ci + 2, b)

            compute(ci, b)

        base0 = chunk_base(0)
        pltpu.sync_copy(s_hbm.at[pl.ds(base0, CH)], si_v.at[0])
        pltpu.sync_copy(p_hbm.at[pl.ds(base0, CH)], pi_v.at[0])
        pltpu.sync_copy(n_hbm.at[pl.ds(base0, CH)], ni_v.at[0])
        gat_issue(0)
        idx_issue(1, 1)

        def body(j, carry):
            step(2 * j, 0, 1)
            step(2 * j + 1, 1, 0)
            return carry

        lax.fori_loop(0, n_my // 2, body, 0)
        out_wait(0)
        out_wait(1)

    return k(A, Bm, s_idx, p_idx, n_idx, w_vec, bout)


def kernel(edge_index, ef, bt, s, p, n,
           W_time, b_time, W_edge, b_edge,
           W_src, b_src, W_dst, b_dst, W_out, b_out,
           neg_samples=1):
    H = W_src.shape[0]
    F = ef.shape[1]
    B = s.shape[0]
    E = bt.shape[0]

    dst = edge_index[1]
    W_ef = W_edge[:F]
    W_tf = W_edge[F:]

    e_quantum = N_WORKERS * 2 * CH
    E_pad = ((E + e_quantum - 1) // e_quantum) * e_quantum
    dst_p = jnp.pad(dst, (0, E_pad - E), constant_values=N_NODES)

    cf = _edge_messages(bt, ef.T, W_time, b_time.reshape(1, H),
                        W_ef, W_tf, b_edge.reshape(1, H),
                        tile_e=2048, e_pad=E_pad)

    zeros = jnp.zeros((N_PAD, H), jnp.float32)
    hp = _segment_sum_sc(cf, dst_p, zeros)

    A, Bm = _node_embeddings(hp.reshape(2, N_PAD, H),
                             W_src, b_src.reshape(1, H),
                             W_dst, b_dst.reshape(1, H), tile_n=2048)

    quantum = N_WORKERS * CH
    Bpad = ((B + quantum - 1) // quantum) * quantum
    pad = Bpad - B
    sp = jnp.pad(s, (0, pad))
    pp = jnp.pad(p, (0, pad))
    np_ = jnp.pad(n, (0, pad))

    posf, negf = _predict_sc(A, Bm, sp, pp, np_,
                             W_out.reshape(H), jnp.broadcast_to(b_out, (LANES,)))
    return posf[:B, None], negf[:B, None]
